# baseline (speedup 1.0000x reference)
import functools

import jax
import jax.numpy as jnp
from jax import lax
from jax.experimental import pallas as pl
from jax.experimental.pallas import tpu as pltpu
from jax.experimental.pallas import tpu_sc as plsc

_T = 5
_N = 1024
_B = 64
_D = 16
_H = 100

_NW = 32
_KPW = (_N * _D) // _NW
_CHUNK = 128
_NCH = _KPW // _CHUNK

_NB = 8
_NBL = _N // _NB


def _leaky(x):
    return jnp.maximum(x, 0.01 * x)


@functools.partial(
    pl.kernel,
    mesh=plsc.VectorSubcoreMesh(core_axis_name="c", subcore_axis_name="s"),
    out_type=jax.ShapeDtypeStruct((_N * _D, _B), jnp.bfloat16),
    scratch_types=[
        pltpu.VMEM((_NCH, _CHUNK), jnp.int32),
        pltpu.VMEM((_KPW, _B), jnp.bfloat16),
    ]
    + [pltpu.SemaphoreType.DMA] * (2 * _NCH),
    compiler_params=pltpu.CompilerParams(use_tc_tiling_on_sc=False),
)
def _gather_sc(table_hbm, idx_hbm, out_hbm, idx_v, rows_v, *sems):
    wid = lax.axis_index("s") * 2 + lax.axis_index("c")
    pltpu.sync_copy(idx_hbm.at[wid], idx_v)
    cps = [
        pltpu.async_copy(
            table_hbm.at[idx_v.at[j]],
            rows_v.at[pl.ds(j * _CHUNK, _CHUNK)],
            sems[j],
        )
        for j in range(_NCH)
    ]
    outs = []
    for j in range(_NCH):
        cps[j].wait()
        outs.append(
            pltpu.async_copy(
                rows_v.at[pl.ds(j * _CHUNK, _CHUNK)],
                out_hbm.at[pl.ds(wid * _KPW + j * _CHUNK, _CHUNK)],
                sems[_NCH + j],
            )
        )
    for cp in outs:
        cp.wait()


def _mlp_math(g_ref, w1_ref, b1_ref, w2_ref, b2_ref, w3_ref, nbl=_NBL):
    g = g_ref[...].reshape(nbl, _D, _B)
    h = lax.dot_general(g, w1_ref[...], (((1,), (1,)), ((0,), (0,))),
                        preferred_element_type=jnp.float32)
    h = _leaky(h + b1_ref[...].astype(jnp.float32)[:, None, :]).astype(jnp.bfloat16)
    h = lax.dot_general(h, w2_ref[...], (((2,), (1,)), ((0,), (0,))),
                        preferred_element_type=jnp.float32)
    h = _leaky(h + b2_ref[...].astype(jnp.float32)[:, None, :])
    o = jnp.sum(h * w3_ref[...].astype(jnp.float32)[:, None, :], axis=-1)
    return _leaky(o)


_NBC = 256


def _mlp_body_conv(g_ref, w1v_ref, b1_ref, w2v_ref, b2_ref, w3_ref,
                   outb_ref, w1b_ref, w2b_ref):
    w1b = jnp.transpose(w1v_ref[...].astype(jnp.bfloat16), (2, 1, 0))
    w2b = jnp.transpose(w2v_ref[...].astype(jnp.bfloat16), (2, 0, 1))
    w1b_ref[...] = w1b
    w2b_ref[...] = w2b
    g = g_ref[...].reshape(_NBC, _D, _B)
    h = lax.dot_general(g, w1b, (((1,), (1,)), ((0,), (0,))),
                        preferred_element_type=jnp.float32)
    h = _leaky(h + b1_ref[...].astype(jnp.float32)[:, None, :]).astype(jnp.bfloat16)
    h = lax.dot_general(h, w2b, (((2,), (1,)), ((0,), (0,))),
                        preferred_element_type=jnp.float32)
    h = _leaky(h + b2_ref[...].astype(jnp.float32)[:, None, :])
    o = jnp.sum(h * w3_ref[...].astype(jnp.float32)[:, None, :], axis=-1)
    outb_ref[...] = _leaky(o).astype(jnp.bfloat16)


def _mlp_conv(g, W1v, b1b, W2v, b2b, W3s):
    return pl.pallas_call(
        _mlp_body_conv,
        grid=(_N // _NBC,),
        in_specs=[
            pl.BlockSpec((_NBC * _D, _B), lambda i: (i, 0)),
            pl.BlockSpec((_H, _D, _NBC), lambda i: (0, 0, i)),
            pl.BlockSpec((_NBC, _H), lambda i: (i, 0)),
            pl.BlockSpec((_H, _H, _NBC), lambda i: (0, 0, i)),
            pl.BlockSpec((_NBC, _H), lambda i: (i, 0)),
            pl.BlockSpec((_NBC, _H), lambda i: (i, 0)),
        ],
        out_specs=[
            pl.BlockSpec((_NBC, _B), lambda i: (i, 0)),
            pl.BlockSpec((_NBC, _D, _H), lambda i: (i, 0, 0)),
            pl.BlockSpec((_NBC, _H, _H), lambda i: (i, 0, 0)),
        ],
        out_shape=[
            jax.ShapeDtypeStruct((_N, _B), jnp.bfloat16),
            jax.ShapeDtypeStruct((_N, _D, _H), jnp.bfloat16),
            jax.ShapeDtypeStruct((_N, _H, _H), jnp.bfloat16),
        ],
    )(g, W1v, b1b, W2v, b2b, W3s)


def _mlp_body_bf(g_ref, w1_ref, b1_ref, w2_ref, b2_ref, w3_ref, outb_ref):
    o = _mlp_math(g_ref, w1_ref, b1_ref, w2_ref, b2_ref, w3_ref)
    outb_ref[...] = o.astype(jnp.bfloat16)


def _mlp_body_last(g_ref, w1_ref, b1_ref, w2_ref, b2_ref, w3_ref, out_ref):
    o = _mlp_math(g_ref, w1_ref, b1_ref, w2_ref, b2_ref, w3_ref, nbl=2 * _NBL)
    out_ref[...] = o.T


_W1SPEC = pl.BlockSpec((_NBL, _D, _H), lambda i: (i, 0, 0))
_W2SPEC = pl.BlockSpec((_NBL, _H, _H), lambda i: (i, 0, 0))
_VSPEC = pl.BlockSpec((_NBL, _H), lambda i: (i, 0))
_GSPEC = pl.BlockSpec((_NBL * _D, _B), lambda i: (i, 0))
_OSPEC = pl.BlockSpec((_NBL, _B), lambda i: (i, 0))

_IN_SPECS = [_GSPEC, _W1SPEC, _VSPEC, _W2SPEC, _VSPEC, _VSPEC]


def _mlp_bf(g, W1b, b1, W2b, b2, W3s):
    return pl.pallas_call(
        _mlp_body_bf,
        grid=(_NB,),
        in_specs=_IN_SPECS,
        out_specs=_OSPEC,
        out_shape=jax.ShapeDtypeStruct((_N, _B), jnp.bfloat16),
    )(g, W1b, b1, W2b, b2, W3s)


_IN_SPECS2 = [
    pl.BlockSpec((2 * _NBL * _D, _B), lambda i: (i, 0)),
    pl.BlockSpec((2 * _NBL, _D, _H), lambda i: (i, 0, 0)),
    pl.BlockSpec((2 * _NBL, _H), lambda i: (i, 0)),
    pl.BlockSpec((2 * _NBL, _H, _H), lambda i: (i, 0, 0)),
    pl.BlockSpec((2 * _NBL, _H), lambda i: (i, 0)),
    pl.BlockSpec((2 * _NBL, _H), lambda i: (i, 0)),
]


def _mlp_last(g, W1b, b1, W2b, b2, W3s):
    return pl.pallas_call(
        _mlp_body_last,
        grid=(_NB // 2,),
        in_specs=_IN_SPECS2,
        out_specs=pl.BlockSpec((_B, 2 * _NBL), lambda i: (0, i)),
        out_shape=jax.ShapeDtypeStruct((_B, _N), jnp.float32),
    )(g, W1b, b1, W2b, b2, W3s)


def kernel(state, pred_idx, W1, b1, W2, b2, W3):
    sb = state.T.astype(jnp.bfloat16)
    idx3 = pred_idx.reshape(_NW, _NCH, _CHUNK)
    g = _gather_sc(sb, idx3)
    W1v = jnp.transpose(W1, (2, 1, 0))
    W2v = jnp.transpose(W2, (1, 2, 0))
    b1b = b1.astype(jnp.bfloat16)
    b2b = b2.astype(jnp.bfloat16)
    W3s = W3[:, :, 0].astype(jnp.bfloat16)
    sb, W1b, W2b = _mlp_conv(g, W1v, b1b, W2v, b2b, W3s)
    for _ in range(_T - 2):
        g = _gather_sc(sb, idx3)
        sb = _mlp_bf(g, W1b, b1b, W2b, b2b, W3s)
    g = _gather_sc(sb, idx3)
    return _mlp_last(g, W1b, b1b, W2b, b2b, W3s)


# speedup vs baseline: 1.0155x; 1.0155x over previous
import functools

import jax
import jax.numpy as jnp
from jax import lax
from jax.experimental import pallas as pl
from jax.experimental.pallas import tpu as pltpu
from jax.experimental.pallas import tpu_sc as plsc

_T = 5
_N = 1024
_B = 64
_D = 16
_H = 100

_NW = 32
_KPW = (_N * _D) // _NW
_CHUNK = 128
_NCH = _KPW // _CHUNK

_NB = 8
_NBL = _N // _NB


def _leaky(x):
    return jnp.maximum(x, 0.01 * x)


@functools.partial(
    pl.kernel,
    mesh=plsc.VectorSubcoreMesh(core_axis_name="c", subcore_axis_name="s"),
    out_type=jax.ShapeDtypeStruct((_N * _D, _B), jnp.bfloat16),
    scratch_types=[
        pltpu.VMEM((_NCH, _CHUNK), jnp.int32),
        pltpu.VMEM((_KPW, _B), jnp.bfloat16),
    ]
    + [pltpu.SemaphoreType.DMA] * (2 * _NCH),
    compiler_params=pltpu.CompilerParams(use_tc_tiling_on_sc=False),
)
def _gather_sc(table_hbm, idx_hbm, out_hbm, idx_v, rows_v, *sems):
    wid = lax.axis_index("s") * 2 + lax.axis_index("c")
    pltpu.sync_copy(idx_hbm.at[wid], idx_v)
    cps = [
        pltpu.async_copy(
            table_hbm.at[idx_v.at[j]],
            rows_v.at[pl.ds(j * _CHUNK, _CHUNK)],
            sems[j],
        )
        for j in range(_NCH)
    ]
    outs = []
    for j in range(_NCH):
        cps[j].wait()
        outs.append(
            pltpu.async_copy(
                rows_v.at[pl.ds(j * _CHUNK, _CHUNK)],
                out_hbm.at[pl.ds(wid * _KPW + j * _CHUNK, _CHUNK)],
                sems[_NCH + j],
            )
        )
    for cp in outs:
        cp.wait()


def _mlp_math(g_ref, w1_ref, b1_ref, w2_ref, b2_ref, w3_ref, nbl=_NBL):
    g = g_ref[...].reshape(nbl, _D, _B)
    h = lax.dot_general(g, w1_ref[...], (((1,), (1,)), ((0,), (0,))),
                        preferred_element_type=jnp.float32)
    h = _leaky(h + b1_ref[...].astype(jnp.float32)[:, None, :]).astype(jnp.bfloat16)
    h = lax.dot_general(h, w2_ref[...], (((2,), (1,)), ((0,), (0,))),
                        preferred_element_type=jnp.float32)
    h = _leaky(h + b2_ref[...].astype(jnp.float32)[:, None, :])
    o = jnp.sum(h * w3_ref[...].astype(jnp.float32)[:, None, :], axis=-1)
    return _leaky(o)


_NBC = 128


def _mlp_body_conv(g_ref, w1v_ref, b1_ref, w2v_ref, b2_ref, w3_ref,
                   outb_ref, w1b_ref, w2b_ref):
    w1b = jnp.transpose(w1v_ref[...].astype(jnp.bfloat16), (2, 1, 0))
    w2b = jnp.transpose(w2v_ref[...].astype(jnp.bfloat16), (2, 0, 1))
    w1b_ref[...] = w1b
    w2b_ref[...] = w2b
    g = g_ref[...].reshape(_NBC, _D, _B)
    h = lax.dot_general(g, w1b, (((1,), (1,)), ((0,), (0,))),
                        preferred_element_type=jnp.float32)
    h = _leaky(h + b1_ref[...].astype(jnp.float32)[:, None, :]).astype(jnp.bfloat16)
    h = lax.dot_general(h, w2b, (((2,), (1,)), ((0,), (0,))),
                        preferred_element_type=jnp.float32)
    h = _leaky(h + b2_ref[...].astype(jnp.float32)[:, None, :])
    o = jnp.sum(h * w3_ref[...].astype(jnp.float32)[:, None, :], axis=-1)
    outb_ref[...] = _leaky(o).astype(jnp.bfloat16)


def _mlp_conv(g, W1v, b1b, W2v, b2b, W3s):
    return pl.pallas_call(
        _mlp_body_conv,
        grid=(_N // _NBC,),
        in_specs=[
            pl.BlockSpec((_NBC * _D, _B), lambda i: (i, 0)),
            pl.BlockSpec((_H, _D, _NBC), lambda i: (0, 0, i)),
            pl.BlockSpec((_NBC, _H), lambda i: (i, 0)),
            pl.BlockSpec((_H, _H, _NBC), lambda i: (0, 0, i)),
            pl.BlockSpec((_NBC, _H), lambda i: (i, 0)),
            pl.BlockSpec((_NBC, _H), lambda i: (i, 0)),
        ],
        out_specs=[
            pl.BlockSpec((_NBC, _B), lambda i: (i, 0)),
            pl.BlockSpec((_NBC, _D, _H), lambda i: (i, 0, 0)),
            pl.BlockSpec((_NBC, _H, _H), lambda i: (i, 0, 0)),
        ],
        out_shape=[
            jax.ShapeDtypeStruct((_N, _B), jnp.bfloat16),
            jax.ShapeDtypeStruct((_N, _D, _H), jnp.bfloat16),
            jax.ShapeDtypeStruct((_N, _H, _H), jnp.bfloat16),
        ],
    )(g, W1v, b1b, W2v, b2b, W3s)


def _mlp_body_bf(g_ref, w1_ref, b1_ref, w2_ref, b2_ref, w3_ref, outb_ref):
    o = _mlp_math(g_ref, w1_ref, b1_ref, w2_ref, b2_ref, w3_ref)
    outb_ref[...] = o.astype(jnp.bfloat16)


def _mlp_body_last(g_ref, w1_ref, b1_ref, w2_ref, b2_ref, w3_ref, out_ref):
    o = _mlp_math(g_ref, w1_ref, b1_ref, w2_ref, b2_ref, w3_ref, nbl=2 * _NBL)
    out_ref[...] = o.T


_W1SPEC = pl.BlockSpec((_NBL, _D, _H), lambda i: (i, 0, 0))
_W2SPEC = pl.BlockSpec((_NBL, _H, _H), lambda i: (i, 0, 0))
_VSPEC = pl.BlockSpec((_NBL, _H), lambda i: (i, 0))
_GSPEC = pl.BlockSpec((_NBL * _D, _B), lambda i: (i, 0))
_OSPEC = pl.BlockSpec((_NBL, _B), lambda i: (i, 0))

_IN_SPECS = [_GSPEC, _W1SPEC, _VSPEC, _W2SPEC, _VSPEC, _VSPEC]


def _mlp_bf(g, W1b, b1, W2b, b2, W3s):
    return pl.pallas_call(
        _mlp_body_bf,
        grid=(_NB,),
        in_specs=_IN_SPECS,
        out_specs=_OSPEC,
        out_shape=jax.ShapeDtypeStruct((_N, _B), jnp.bfloat16),
    )(g, W1b, b1, W2b, b2, W3s)


_IN_SPECS2 = [
    pl.BlockSpec((2 * _NBL * _D, _B), lambda i: (i, 0)),
    pl.BlockSpec((2 * _NBL, _D, _H), lambda i: (i, 0, 0)),
    pl.BlockSpec((2 * _NBL, _H), lambda i: (i, 0)),
    pl.BlockSpec((2 * _NBL, _H, _H), lambda i: (i, 0, 0)),
    pl.BlockSpec((2 * _NBL, _H), lambda i: (i, 0)),
    pl.BlockSpec((2 * _NBL, _H), lambda i: (i, 0)),
]


def _mlp_last(g, W1b, b1, W2b, b2, W3s):
    return pl.pallas_call(
        _mlp_body_last,
        grid=(_NB // 2,),
        in_specs=_IN_SPECS2,
        out_specs=pl.BlockSpec((_B, 2 * _NBL), lambda i: (0, i)),
        out_shape=jax.ShapeDtypeStruct((_B, _N), jnp.float32),
    )(g, W1b, b1, W2b, b2, W3s)


def kernel(state, pred_idx, W1, b1, W2, b2, W3):
    sb = state.T.astype(jnp.bfloat16)
    idx3 = pred_idx.reshape(_NW, _NCH, _CHUNK)
    g = _gather_sc(sb, idx3)
    W1v = jnp.transpose(W1, (2, 1, 0))
    W2v = jnp.transpose(W2, (1, 2, 0))
    b1b = b1.astype(jnp.bfloat16)
    b2b = b2.astype(jnp.bfloat16)
    W3s = W3[:, :, 0].astype(jnp.bfloat16)
    sb, W1b, W2b = _mlp_conv(g, W1v, b1b, W2v, b2b, W3s)
    for _ in range(_T - 2):
        g = _gather_sc(sb, idx3)
        sb = _mlp_bf(g, W1b, b1b, W2b, b2b, W3s)
    g = _gather_sc(sb, idx3)
    return _mlp_last(g, W1b, b1b, W2b, b2b, W3s)
